# baseline (speedup 1.0000x reference)
import functools

import jax
import jax.numpy as jnp
import numpy as np
from jax import lax
from jax.experimental import pallas as pl
from jax.experimental.pallas import tpu as pltpu
from jax.experimental.pallas import tpu_sc as plsc

N = 10000
NP = 10240
D = 2048
E = 40000
EP = 81920
NC = 2
NS = 16
CW = 128
NCHUNK = D // CW
TROWS = NP // NS
EPT = EP // NS
NB = EPT // 128
DPT = EP // (NC * NS)
DB = DPT // 128

SPLITS = (2, 2, 2, 10)
OFFS = (0, 2, 4, 6)
NSPLIT = len(SPLITS)

_mesh = plsc.VectorSubcoreMesh(core_axis_name="c", subcore_axis_name="s")


@functools.partial(
    pl.kernel,
    out_type=jax.ShapeDtypeStruct((NC, NP, 128), jnp.float32),
    mesh=_mesh,
    scratch_types=[
        pltpu.VMEM((DB, 128), jnp.int32),
        pltpu.VMEM((128, 128), jnp.float32),
        pltpu.VMEM_SHARED((NP, 128), jnp.float32),
    ],
)
def _deg_kernel(rows_hbm, zeros_hbm, ones_hbm, out_hbm, idx_v, ones_v, acc):
    cid = lax.axis_index("c")
    sid = lax.axis_index("s")
    r0 = sid * TROWS
    pltpu.sync_copy(zeros_hbm.at[pl.ds(r0, TROWS)], acc.at[pl.ds(r0, TROWS)])
    pltpu.sync_copy(ones_hbm, ones_v)
    pltpu.sync_copy(rows_hbm.at[cid, sid], idx_v)
    plsc.subcore_barrier()

    @pl.loop(jnp.int32(0), jnp.int32(DB))
    def _deg_body(b):
        pltpu.sync_copy(ones_v, acc.at[idx_v.at[b]], add=True)
    plsc.subcore_barrier()
    pltpu.sync_copy(acc.at[pl.ds(r0, TROWS)], out_hbm.at[cid, pl.ds(r0, TROWS)])


def _make_agg_kernel(ng):
    @functools.partial(
        pl.kernel,
        out_type=jax.ShapeDtypeStruct((NP, ng * CW), jnp.float32),
        mesh=_mesh,
        scratch_types=[
            pltpu.VMEM((NB, 128), jnp.int32),
            pltpu.VMEM((NB, 128), jnp.int32),
            pltpu.VMEM((128, CW), jnp.float32),
            pltpu.VMEM((128, CW), jnp.float32),
            pltpu.VMEM_SHARED((NP, CW), jnp.float32),
            pltpu.SemaphoreType.DMA,
            pltpu.SemaphoreType.DMA,
            pltpu.SemaphoreType.DMA,
            pltpu.SemaphoreType.DMA,
        ],
    )
    def _agg(s_hbm, rows_hbm, cols_hbm, out_hbm,
             rows_v, cols_v, gb0, gb1, acc, gs0, gs1, ss0, ss1):
        cid = lax.axis_index("c")
        sid = lax.axis_index("s")
        r0 = sid * TROWS
        pltpu.sync_copy(rows_hbm.at[sid], rows_v)

        for k in range(ng // NC):
            kk = cid * (ng // NC) + k
            pltpu.sync_copy(s_hbm.at[pl.ds(kk * NP + r0, TROWS)],
                            acc.at[pl.ds(r0, TROWS)])
            pltpu.sync_copy(cols_hbm.at[kk, sid], cols_v)
            plsc.subcore_barrier()

            gbufs = (gb0, gb1)
            gsems = (gs0, gs1)
            ssems = (ss0, ss1)
            for j in range(2):
                pltpu.async_copy(s_hbm.at[cols_v.at[jnp.int32(j)]],
                                 gbufs[j], gsems[j])

            @pl.loop(jnp.int32(0), jnp.int32(NB), step=jnp.int32(2))
            def _edge_body(b):
                for j in range(2):
                    pltpu.make_async_copy(
                        s_hbm.at[cols_v.at[b]], gbufs[j], gsems[j]).wait()
                    pltpu.async_copy(gbufs[j], acc.at[rows_v.at[b + j]],
                                     ssems[j], add=True)

                @pl.when(b + 2 < NB)
                def _():
                    for j in range(2):
                        pltpu.make_async_copy(
                            gbufs[j], acc.at[rows_v.at[b]], ssems[j]).wait()
                        pltpu.async_copy(s_hbm.at[cols_v.at[b + 2 + j]],
                                         gbufs[j], gsems[j])

            for j in range(2):
                pltpu.make_async_copy(
                    gbufs[j], acc.at[rows_v.at[jnp.int32(0)]], ssems[j]).wait()
            plsc.subcore_barrier()
            pltpu.sync_copy(acc.at[pl.ds(r0, TROWS)],
                            out_hbm.at[pl.ds(r0, TROWS), pl.ds(kk * CW, CW)])

    return _agg


_agg_kernels = {ng: _make_agg_kernel(ng) for ng in sorted(set(SPLITS))}


_i0 = np.int32(0)


def _mm3_body(xh_ref, xl_ref, wh_ref, wl_ref, o_ref):
    xh = xh_ref[...]
    acc = jnp.dot(xh, wh_ref[...], preferred_element_type=jnp.float32)
    acc = acc + jnp.dot(xh, wl_ref[...], preferred_element_type=jnp.float32)
    acc = acc + jnp.dot(xl_ref[...], wh_ref[...],
                        preferred_element_type=jnp.float32)
    o_ref[0] = acc


def _mm_group(xh, xl, wh, wl, base, ng):
    bm = 1024
    bb = np.int32(base)
    return pl.pallas_call(
        _mm3_body,
        grid=(NP // bm, ng),
        in_specs=[
            pl.BlockSpec((bm, D), lambda i, k: (i, _i0)),
            pl.BlockSpec((bm, D), lambda i, k: (i, _i0)),
            pl.BlockSpec((D, CW), lambda i, k: (_i0, k + bb)),
            pl.BlockSpec((D, CW), lambda i, k: (_i0, k + bb)),
        ],
        out_specs=pl.BlockSpec((1, bm, CW), lambda i, k: (k, i, _i0)),
        out_shape=jax.ShapeDtypeStruct((ng, NP, CW), jnp.float32),
    )(xh, xl, wh, wl)


def _mm_groups_body(h0h, h0l, h1h, h1l, h2h, h2l, h3h, h3l,
                    wh_ref, wl_ref, o_ref):
    hs = ((h0h, h0l), (h1h, h1l), (h2h, h2l), (h3h, h3l))
    wh = wh_ref[...]
    wl = wl_ref[...]
    acc = jnp.zeros((h0h.shape[0], CW), jnp.float32)
    for g, (hh_ref, hl_ref) in enumerate(hs):
        lo = OFFS[g] * CW
        hi = (OFFS[g] + SPLITS[g]) * CW
        whg = wh[lo:hi, :]
        wlg = wl[lo:hi, :]
        xh = hh_ref[...]
        acc = acc + jnp.dot(xh, whg, preferred_element_type=jnp.float32)
        acc = acc + jnp.dot(xh, wlg, preferred_element_type=jnp.float32)
        acc = acc + jnp.dot(hl_ref[...], whg,
                            preferred_element_type=jnp.float32)
    o_ref[0] = acc


def _mm_groups(hs, wh, wl, base, ng):
    bm = 1024
    bb = np.int32(base)
    hspecs = []
    for g in range(NSPLIT):
        for _ in range(2):
            hspecs.append(
                pl.BlockSpec((bm, SPLITS[g] * CW), lambda i, k: (i, _i0)))
    return pl.pallas_call(
        _mm_groups_body,
        grid=(NP // bm, ng),
        in_specs=hspecs + [
            pl.BlockSpec((D, CW), lambda i, k: (_i0, k + bb)),
            pl.BlockSpec((D, CW), lambda i, k: (_i0, k + bb)),
        ],
        out_specs=pl.BlockSpec((1, bm, CW), lambda i, k: (k, i, _i0)),
        out_shape=jax.ShapeDtypeStruct((ng, NP, CW), jnp.float32),
    )(*hs, wh, wl)


def _split_bf16(a):
    hi = a.astype(jnp.bfloat16)
    lo = (a - hi.astype(jnp.float32)).astype(jnp.bfloat16)
    return hi, lo


def _pointwise_group(u, pt, b, width):
    bm = 1024

    def body(u_ref, pt_ref, b_ref, hi_ref, lo_ref):
        inv = 1.0 / (1.0 + pt_ref[:, 0:1] + pt_ref[:, 1:2])
        val = jnp.maximum(u_ref[...] * inv + b_ref[...], 0.0)
        hi = val.astype(jnp.bfloat16)
        hi_ref[...] = hi
        lo_ref[...] = (val - hi.astype(jnp.float32)).astype(jnp.bfloat16)

    return pl.pallas_call(
        body,
        grid=(NP // bm,),
        in_specs=[
            pl.BlockSpec((bm, width), lambda i: (i, _i0)),
            pl.BlockSpec((bm, 2), lambda i: (i, _i0)),
            pl.BlockSpec((1, width), lambda i: (_i0, _i0)),
        ],
        out_specs=[
            pl.BlockSpec((bm, width), lambda i: (i, _i0)),
            pl.BlockSpec((bm, width), lambda i: (i, _i0)),
        ],
        out_shape=[
            jax.ShapeDtypeStruct((NP, width), jnp.bfloat16),
            jax.ShapeDtypeStruct((NP, width), jnp.bfloat16),
        ],
    )(u, pt, b)


def _pointwise_final(us, pt, b):
    bm = 1024

    def body(u0, u1, u2, u3, pt_ref, b_ref, o_ref):
        inv = 1.0 / (1.0 + pt_ref[:, 0:1] + pt_ref[:, 1:2])
        for g, u_ref in enumerate((u0, u1, u2, u3)):
            lo = OFFS[g] * CW
            hi = (OFFS[g] + SPLITS[g]) * CW
            o_ref[:, lo:hi] = u_ref[...] * inv + b_ref[:, lo:hi]

    return pl.pallas_call(
        body,
        grid=(NP // bm,),
        in_specs=[pl.BlockSpec((bm, SPLITS[g] * CW), lambda i: (i, _i0))
                  for g in range(NSPLIT)] + [
            pl.BlockSpec((bm, 2), lambda i: (i, _i0)),
            pl.BlockSpec((1, D), lambda i: (_i0, _i0)),
        ],
        out_specs=pl.BlockSpec((bm, D), lambda i: (i, _i0)),
        out_shape=jax.ShapeDtypeStruct((NP, D), jnp.float32),
    )(*us, pt, b)


def kernel(x, edge_index, W1, b1, W2, b2):
    src = edge_index[0].astype(jnp.int32)
    dst = edge_index[1].astype(jnp.int32)
    rows = jnp.concatenate([src, dst])
    cols = jnp.concatenate([dst, src])
    pad = jnp.full((EP - 2 * E,), NP - 1, jnp.int32)
    rows_p = jnp.concatenate([rows, pad])
    cols_p = jnp.concatenate([cols, pad])

    rows3 = rows_p.reshape(NS, NB, 128)
    cols1 = cols_p.reshape(1, NS, NB, 128)
    colsL = {
        ng: cols1 + (jnp.arange(ng, dtype=jnp.int32) * NP)[:, None, None, None]
        for ng in sorted(set(SPLITS))
    }
    rows_deg = rows_p.reshape(NC, NS, DB, 128)

    x_p = jnp.zeros((NP, D), jnp.float32).at[:N].set(x.astype(jnp.float32))
    zeros16 = jnp.zeros((NP, 128), jnp.float32)
    ones16 = jnp.ones((128, 128), jnp.float32)

    partials = _deg_kernel(rows_deg, zeros16, ones16)
    pt = partials[:, :, 0].T
    b1r = b1.astype(jnp.float32).reshape(1, D)
    b2r = b2.astype(jnp.float32).reshape(1, D)
    xh, xl = _split_bf16(x_p)
    w1h, w1l = _split_bf16(W1.astype(jnp.float32))
    w2h, w2l = _split_bf16(W2.astype(jnp.float32))

    hs = []
    u1s = []
    for g in range(NSPLIT):
        ng = SPLITS[g]
        s1g = _mm_group(xh, xl, w1h, w1l, OFFS[g], ng)
        u1s.append(_agg_kernels[ng](s1g.reshape(ng * NP, CW), rows3,
                                    colsL[ng]))
    for g in range(NSPLIT):
        lo = OFFS[g] * CW
        hi = (OFFS[g] + SPLITS[g]) * CW
        hh_g, hl_g = _pointwise_group(u1s[g], pt, b1r[:, lo:hi],
                                      SPLITS[g] * CW)
        hs.extend((hh_g, hl_g))

    u2s = []
    for g in range(NSPLIT):
        ng = SPLITS[g]
        s2g = _mm_groups(hs, w2h, w2l, OFFS[g], ng)
        u2s.append(_agg_kernels[ng](s2g.reshape(ng * NP, CW), rows3,
                                    colsL[ng]))
    out = _pointwise_final(u2s, pt, b2r)
    return out[:N]


# speedup vs baseline: 1.0425x; 1.0425x over previous
import functools

import jax
import jax.numpy as jnp
import numpy as np
from jax import lax
from jax.experimental import pallas as pl
from jax.experimental.pallas import tpu as pltpu
from jax.experimental.pallas import tpu_sc as plsc

N = 10000
NP = 10240
D = 2048
E = 40000
EP = 81920
NC = 2
NS = 16
CW = 128
NCHUNK = D // CW
TROWS = NP // NS
EPT = EP // NS
NB = EPT // 128
DPT = EP // (NC * NS)
DB = DPT // 128

SPLITS = (2, 2, 6, 6)
OFFS = (0, 2, 4, 10)
NSPLIT = len(SPLITS)

_mesh = plsc.VectorSubcoreMesh(core_axis_name="c", subcore_axis_name="s")


@functools.partial(
    pl.kernel,
    out_type=jax.ShapeDtypeStruct((NC, NP, 128), jnp.float32),
    mesh=_mesh,
    scratch_types=[
        pltpu.VMEM((DB, 128), jnp.int32),
        pltpu.VMEM((128, 128), jnp.float32),
        pltpu.VMEM_SHARED((NP, 128), jnp.float32),
    ],
)
def _deg_kernel(rows_hbm, zeros_hbm, ones_hbm, out_hbm, idx_v, ones_v, acc):
    cid = lax.axis_index("c")
    sid = lax.axis_index("s")
    r0 = sid * TROWS
    pltpu.sync_copy(zeros_hbm.at[pl.ds(r0, TROWS)], acc.at[pl.ds(r0, TROWS)])
    pltpu.sync_copy(ones_hbm, ones_v)
    pltpu.sync_copy(rows_hbm.at[cid, sid], idx_v)
    plsc.subcore_barrier()

    @pl.loop(jnp.int32(0), jnp.int32(DB))
    def _deg_body(b):
        pltpu.sync_copy(ones_v, acc.at[idx_v.at[b]], add=True)
    plsc.subcore_barrier()
    pltpu.sync_copy(acc.at[pl.ds(r0, TROWS)], out_hbm.at[cid, pl.ds(r0, TROWS)])


def _make_agg_kernel(ng):
    @functools.partial(
        pl.kernel,
        out_type=jax.ShapeDtypeStruct((NP, ng * CW), jnp.float32),
        mesh=_mesh,
        scratch_types=[
            pltpu.VMEM((NB, 128), jnp.int32),
            pltpu.VMEM((NB, 128), jnp.int32),
            pltpu.VMEM((128, CW), jnp.float32),
            pltpu.VMEM((128, CW), jnp.float32),
            pltpu.VMEM_SHARED((NP, CW), jnp.float32),
            pltpu.SemaphoreType.DMA,
            pltpu.SemaphoreType.DMA,
            pltpu.SemaphoreType.DMA,
            pltpu.SemaphoreType.DMA,
        ],
    )
    def _agg(s_hbm, rows_hbm, cols_hbm, out_hbm,
             rows_v, cols_v, gb0, gb1, acc, gs0, gs1, ss0, ss1):
        cid = lax.axis_index("c")
        sid = lax.axis_index("s")
        r0 = sid * TROWS
        pltpu.sync_copy(rows_hbm.at[sid], rows_v)

        for k in range(ng // NC):
            kk = cid * (ng // NC) + k
            pltpu.sync_copy(s_hbm.at[pl.ds(kk * NP + r0, TROWS)],
                            acc.at[pl.ds(r0, TROWS)])
            pltpu.sync_copy(cols_hbm.at[kk, sid], cols_v)
            plsc.subcore_barrier()

            gbufs = (gb0, gb1)
            gsems = (gs0, gs1)
            ssems = (ss0, ss1)
            for j in range(2):
                pltpu.async_copy(s_hbm.at[cols_v.at[jnp.int32(j)]],
                                 gbufs[j], gsems[j])

            @pl.loop(jnp.int32(0), jnp.int32(NB), step=jnp.int32(2))
            def _edge_body(b):
                for j in range(2):
                    pltpu.make_async_copy(
                        s_hbm.at[cols_v.at[b]], gbufs[j], gsems[j]).wait()
                    pltpu.async_copy(gbufs[j], acc.at[rows_v.at[b + j]],
                                     ssems[j], add=True)

                @pl.when(b + 2 < NB)
                def _():
                    for j in range(2):
                        pltpu.make_async_copy(
                            gbufs[j], acc.at[rows_v.at[b]], ssems[j]).wait()
                        pltpu.async_copy(s_hbm.at[cols_v.at[b + 2 + j]],
                                         gbufs[j], gsems[j])

            for j in range(2):
                pltpu.make_async_copy(
                    gbufs[j], acc.at[rows_v.at[jnp.int32(0)]], ssems[j]).wait()
            plsc.subcore_barrier()
            pltpu.sync_copy(acc.at[pl.ds(r0, TROWS)],
                            out_hbm.at[pl.ds(r0, TROWS), pl.ds(kk * CW, CW)])

    return _agg


_agg_kernels = {ng: _make_agg_kernel(ng) for ng in sorted(set(SPLITS))}


_i0 = np.int32(0)


def _mm3_body(xh_ref, xl_ref, wh_ref, wl_ref, o_ref):
    xh = xh_ref[...]
    acc = jnp.dot(xh, wh_ref[...], preferred_element_type=jnp.float32)
    acc = acc + jnp.dot(xh, wl_ref[...], preferred_element_type=jnp.float32)
    acc = acc + jnp.dot(xl_ref[...], wh_ref[...],
                        preferred_element_type=jnp.float32)
    o_ref[0] = acc


def _mm_group(xh, xl, wh, wl, base, ng):
    bm = 1024
    bb = np.int32(base)
    return pl.pallas_call(
        _mm3_body,
        grid=(NP // bm, ng),
        in_specs=[
            pl.BlockSpec((bm, D), lambda i, k: (i, _i0)),
            pl.BlockSpec((bm, D), lambda i, k: (i, _i0)),
            pl.BlockSpec((D, CW), lambda i, k: (_i0, k + bb)),
            pl.BlockSpec((D, CW), lambda i, k: (_i0, k + bb)),
        ],
        out_specs=pl.BlockSpec((1, bm, CW), lambda i, k: (k, i, _i0)),
        out_shape=jax.ShapeDtypeStruct((ng, NP, CW), jnp.float32),
    )(xh, xl, wh, wl)


def _mm_groups_body(h0h, h0l, h1h, h1l, h2h, h2l, h3h, h3l,
                    wh_ref, wl_ref, o_ref):
    hs = ((h0h, h0l), (h1h, h1l), (h2h, h2l), (h3h, h3l))
    wh = wh_ref[...]
    wl = wl_ref[...]
    acc = jnp.zeros((h0h.shape[0], CW), jnp.float32)
    for g, (hh_ref, hl_ref) in enumerate(hs):
        lo = OFFS[g] * CW
        hi = (OFFS[g] + SPLITS[g]) * CW
        whg = wh[lo:hi, :]
        wlg = wl[lo:hi, :]
        xh = hh_ref[...]
        acc = acc + jnp.dot(xh, whg, preferred_element_type=jnp.float32)
        acc = acc + jnp.dot(xh, wlg, preferred_element_type=jnp.float32)
        acc = acc + jnp.dot(hl_ref[...], whg,
                            preferred_element_type=jnp.float32)
    o_ref[0] = acc


def _mm_groups(hs, wh, wl, base, ng):
    bm = 1024
    bb = np.int32(base)
    hspecs = []
    for g in range(NSPLIT):
        for _ in range(2):
            hspecs.append(
                pl.BlockSpec((bm, SPLITS[g] * CW), lambda i, k: (i, _i0)))
    return pl.pallas_call(
        _mm_groups_body,
        grid=(NP // bm, ng),
        in_specs=hspecs + [
            pl.BlockSpec((D, CW), lambda i, k: (_i0, k + bb)),
            pl.BlockSpec((D, CW), lambda i, k: (_i0, k + bb)),
        ],
        out_specs=pl.BlockSpec((1, bm, CW), lambda i, k: (k, i, _i0)),
        out_shape=jax.ShapeDtypeStruct((ng, NP, CW), jnp.float32),
    )(*hs, wh, wl)


def _split_bf16(a):
    hi = a.astype(jnp.bfloat16)
    lo = (a - hi.astype(jnp.float32)).astype(jnp.bfloat16)
    return hi, lo


def _pointwise_group(u, pt, b, width):
    bm = 1024

    def body(u_ref, pt_ref, b_ref, hi_ref, lo_ref):
        inv = 1.0 / (1.0 + pt_ref[:, 0:1] + pt_ref[:, 1:2])
        val = jnp.maximum(u_ref[...] * inv + b_ref[...], 0.0)
        hi = val.astype(jnp.bfloat16)
        hi_ref[...] = hi
        lo_ref[...] = (val - hi.astype(jnp.float32)).astype(jnp.bfloat16)

    return pl.pallas_call(
        body,
        grid=(NP // bm,),
        in_specs=[
            pl.BlockSpec((bm, width), lambda i: (i, _i0)),
            pl.BlockSpec((bm, 2), lambda i: (i, _i0)),
            pl.BlockSpec((1, width), lambda i: (_i0, _i0)),
        ],
        out_specs=[
            pl.BlockSpec((bm, width), lambda i: (i, _i0)),
            pl.BlockSpec((bm, width), lambda i: (i, _i0)),
        ],
        out_shape=[
            jax.ShapeDtypeStruct((NP, width), jnp.bfloat16),
            jax.ShapeDtypeStruct((NP, width), jnp.bfloat16),
        ],
    )(u, pt, b)


def _pointwise_final(us, pt, b):
    bm = 1024

    def body(u0, u1, u2, u3, pt_ref, b_ref, o_ref):
        inv = 1.0 / (1.0 + pt_ref[:, 0:1] + pt_ref[:, 1:2])
        for g, u_ref in enumerate((u0, u1, u2, u3)):
            lo = OFFS[g] * CW
            hi = (OFFS[g] + SPLITS[g]) * CW
            o_ref[:, lo:hi] = u_ref[...] * inv + b_ref[:, lo:hi]

    return pl.pallas_call(
        body,
        grid=(NP // bm,),
        in_specs=[pl.BlockSpec((bm, SPLITS[g] * CW), lambda i: (i, _i0))
                  for g in range(NSPLIT)] + [
            pl.BlockSpec((bm, 2), lambda i: (i, _i0)),
            pl.BlockSpec((1, D), lambda i: (_i0, _i0)),
        ],
        out_specs=pl.BlockSpec((bm, D), lambda i: (i, _i0)),
        out_shape=jax.ShapeDtypeStruct((NP, D), jnp.float32),
    )(*us, pt, b)


def kernel(x, edge_index, W1, b1, W2, b2):
    src = edge_index[0].astype(jnp.int32)
    dst = edge_index[1].astype(jnp.int32)
    rows = jnp.concatenate([src, dst])
    cols = jnp.concatenate([dst, src])
    pad = jnp.full((EP - 2 * E,), NP - 1, jnp.int32)
    rows_p = jnp.concatenate([rows, pad])
    cols_p = jnp.concatenate([cols, pad])

    rows3 = rows_p.reshape(NS, NB, 128)
    cols1 = cols_p.reshape(1, NS, NB, 128)
    colsL = {
        ng: cols1 + (jnp.arange(ng, dtype=jnp.int32) * NP)[:, None, None, None]
        for ng in sorted(set(SPLITS))
    }
    rows_deg = rows_p.reshape(NC, NS, DB, 128)

    x_p = jnp.zeros((NP, D), jnp.float32).at[:N].set(x.astype(jnp.float32))
    zeros16 = jnp.zeros((NP, 128), jnp.float32)
    ones16 = jnp.ones((128, 128), jnp.float32)

    partials = _deg_kernel(rows_deg, zeros16, ones16)
    pt = partials[:, :, 0].T
    b1r = b1.astype(jnp.float32).reshape(1, D)
    b2r = b2.astype(jnp.float32).reshape(1, D)
    xh, xl = _split_bf16(x_p)
    w1h, w1l = _split_bf16(W1.astype(jnp.float32))
    w2h, w2l = _split_bf16(W2.astype(jnp.float32))

    hs = []
    u1s = []
    for g in range(NSPLIT):
        ng = SPLITS[g]
        s1g = _mm_group(xh, xl, w1h, w1l, OFFS[g], ng)
        u1s.append(_agg_kernels[ng](s1g.reshape(ng * NP, CW), rows3,
                                    colsL[ng]))
    for g in range(NSPLIT):
        lo = OFFS[g] * CW
        hi = (OFFS[g] + SPLITS[g]) * CW
        hh_g, hl_g = _pointwise_group(u1s[g], pt, b1r[:, lo:hi],
                                      SPLITS[g] * CW)
        hs.extend((hh_g, hl_g))

    u2s = []
    for g in range(NSPLIT):
        ng = SPLITS[g]
        s2g = _mm_groups(hs, w2h, w2l, OFFS[g], ng)
        u2s.append(_agg_kernels[ng](s2g.reshape(ng * NP, CW), rows3,
                                    colsL[ng]))
    out = _pointwise_final(u2s, pt, b2r)
    return out[:N]
